# baseline (speedup 1.0000x reference)
import jax
import jax.numpy as jnp
from jax.experimental import pallas as pl
from jax.experimental.pallas import tpu as pltpu

EPS = 1e-06
NUM_GROUPS = 4
CH = 512
NCH = 16
D = 1024


def _body(x_hbm, tt_ref, g_ref, b_ref, o_hbm,
          in0, in1, ot0, ot1, isem, osem):
    ins = [in0, in1]
    ots = [ot0, ot1]
    g = g_ref[...]
    b = b_ref[...]

    def in_copy(i):
        return pltpu.make_async_copy(
            x_hbm.at[pl.ds(i * CH, CH)], ins[i % 2], isem)

    def out_copy(i):
        return pltpu.make_async_copy(
            ots[i % 2], o_hbm.at[pl.ds(i * CH, CH)], osem)

    in_copy(0).start()
    for i in range(NCH):
        if i + 1 < NCH:
            in_copy(i + 1).start()
        in_copy(i).wait()
        if i >= 2:
            out_copy(i - 2).wait()
        x = ins[i % 2][...]
        tt = tt_ref[pl.ds(i * CH, CH), :]
        mean = jnp.mean(x, axis=1, keepdims=True)
        xc = x - mean
        var = jnp.mean(xc * xc, axis=1, keepdims=True)
        inv = jax.lax.rsqrt(var + EPS)
        onehot = (tt == jnp.arange(NUM_GROUPS)[None, :]).astype(jnp.float32)
        gg = jax.lax.dot(onehot, g)
        bb = jax.lax.dot(onehot, b)
        ots[i % 2][...] = xc * (inv * gg) + bb
        out_copy(i).start()
    out_copy(NCH - 2).wait()
    out_copy(NCH - 1).wait()


def kernel(x, token_types, gamma, beta):
    B, S, D_ = x.shape
    n_tok = B * S
    x2 = x.reshape(n_tok, D_)
    tt2 = token_types.reshape(n_tok, 1).astype(jnp.int32)
    out = pl.pallas_call(
        _body,
        in_specs=[
            pl.BlockSpec(memory_space=pl.ANY),
            pl.BlockSpec(memory_space=pltpu.VMEM),
            pl.BlockSpec(memory_space=pltpu.VMEM),
            pl.BlockSpec(memory_space=pltpu.VMEM),
        ],
        out_specs=pl.BlockSpec(memory_space=pl.ANY),
        out_shape=jax.ShapeDtypeStruct((n_tok, D_), x.dtype),
        scratch_shapes=[
            pltpu.VMEM((CH, D), jnp.float32),
            pltpu.VMEM((CH, D), jnp.float32),
            pltpu.VMEM((CH, D), jnp.float32),
            pltpu.VMEM((CH, D), jnp.float32),
            pltpu.SemaphoreType.DMA,
            pltpu.SemaphoreType.DMA,
        ],
    )(x2, tt2, gamma, beta)
    return out.reshape(B, S, D_)


# speedup vs baseline: 1.0984x; 1.0984x over previous
import jax
import jax.numpy as jnp
from jax.experimental import pallas as pl
from jax.experimental.pallas import tpu as pltpu

EPS = 1e-06
NUM_GROUPS = 4
CH = 2048
NCH = 4
D = 1024


def _body(x_hbm, tt_ref, g_ref, b_ref, o_hbm,
          in0, in1, ot0, ot1, isem, osem):
    ins = [in0, in1]
    ots = [ot0, ot1]
    g = g_ref[...]
    b = b_ref[...]

    def in_copy(i):
        return pltpu.make_async_copy(
            x_hbm.at[pl.ds(i * CH, CH)], ins[i % 2], isem)

    def out_copy(i):
        return pltpu.make_async_copy(
            ots[i % 2], o_hbm.at[pl.ds(i * CH, CH)], osem)

    in_copy(0).start()
    for i in range(NCH):
        if i + 1 < NCH:
            in_copy(i + 1).start()
        in_copy(i).wait()
        if i >= 2:
            out_copy(i - 2).wait()
        x = ins[i % 2][...]
        tt = tt_ref[pl.ds(i * CH, CH), :]
        mean = jnp.mean(x, axis=1, keepdims=True)
        xc = x - mean
        var = jnp.mean(xc * xc, axis=1, keepdims=True)
        inv = jax.lax.rsqrt(var + EPS)
        onehot = (tt == jnp.arange(NUM_GROUPS)[None, :]).astype(jnp.float32)
        gg = jax.lax.dot(onehot, g)
        bb = jax.lax.dot(onehot, b)
        ots[i % 2][...] = xc * (inv * gg) + bb
        out_copy(i).start()
    out_copy(NCH - 2).wait()
    out_copy(NCH - 1).wait()


def kernel(x, token_types, gamma, beta):
    B, S, D_ = x.shape
    n_tok = B * S
    x2 = x.reshape(n_tok, D_)
    tt2 = token_types.reshape(n_tok, 1).astype(jnp.int32)
    out = pl.pallas_call(
        _body,
        in_specs=[
            pl.BlockSpec(memory_space=pl.ANY),
            pl.BlockSpec(memory_space=pltpu.VMEM),
            pl.BlockSpec(memory_space=pltpu.VMEM),
            pl.BlockSpec(memory_space=pltpu.VMEM),
        ],
        out_specs=pl.BlockSpec(memory_space=pl.ANY),
        out_shape=jax.ShapeDtypeStruct((n_tok, D_), x.dtype),
        scratch_shapes=[
            pltpu.VMEM((CH, D), jnp.float32),
            pltpu.VMEM((CH, D), jnp.float32),
            pltpu.VMEM((CH, D), jnp.float32),
            pltpu.VMEM((CH, D), jnp.float32),
            pltpu.SemaphoreType.DMA,
            pltpu.SemaphoreType.DMA,
        ],
        compiler_params=pltpu.CompilerParams(vmem_limit_bytes=120 * 1024 * 1024),
    )(x2, tt2, gamma, beta)
    return out.reshape(B, S, D_)
